# baseline (speedup 1.0000x reference)
import functools

import jax
import jax.numpy as jnp
from jax import lax
from jax.experimental import pallas as pl
from jax.experimental.pallas import tpu as pltpu
from jax.experimental.pallas import tpu_sc as plsc

KS = 7
PAD = KS // 2
H = 384
W = 384
KK = KS * KS
TOP_NUM = (H * W * KK) // 10
NB = 2048
BINV = 32.0
NSAMP = 4
NBLK = 8
BROWS = H // NBLK
PH = H + 2 * PAD
PW = 392
CHUNKS = W // 16


def _sc_hist_body(pred_hbm, label_hbm, out_hbm, pv, lv, cnt_v, sum_v):
    cid = lax.axis_index("c")
    sid = lax.axis_index("s")
    wid = cid * 16 + sid
    samp = cid * 2 + sid // 8
    block = sid % 8
    row0 = BROWS * block

    pltpu.sync_copy(pred_hbm.at[samp, pl.ds(row0, BROWS + 2 * PAD)], pv)
    pltpu.sync_copy(label_hbm.at[samp, pl.ds(row0, BROWS + 2 * PAD)], lv)

    zeros16 = jnp.zeros((16,), jnp.float32)

    def zinit(i, carry):
        cnt_v[pl.ds(i * 16, 16)] = zeros16
        sum_v[pl.ds(i * 16, 16)] = zeros16
        return carry

    lax.fori_loop(0, NB // 16, zinit, 0)

    ones = jnp.ones((16,), jnp.float32)
    fzero = jnp.float32(0.0)

    offs = [(di, dj) for di in range(PAD + 1, KS) for dj in range(KS)]
    offs += [(PAD, dj) for dj in range(PAD + 1, KS)]
    GRP = 12
    twos = jnp.full((16,), 2.0, jnp.float32)
    lane = lax.iota(jnp.int32, 16)

    def it(t):
        y = t // CHUNKS
        c0 = (t % CHUNKS) * 16
        g = BROWS * block + y
        colv = lane + c0
        pc = pv[y + PAD, pl.ds(c0 + PAD, 16)]
        lc = lv[y + PAD, pl.ds(c0 + PAD, 16)]

        mcol = {dj: ((colv + (dj - PAD)) >= 0) & ((colv + (dj - PAD)) < W)
                for dj in range(KS)}
        mrow = {di: (g + (di - PAD)) < H for di in range(PAD + 1, KS)}

        for g0 in range(0, len(offs), GRP):
            staged = []
            for di, dj in offs[g0:g0 + GRP]:
                pn = pv[y + di, pl.ds(c0 + dj, 16)]
                ln = lv[y + di, pl.ds(c0 + dj, 16)]
                v = jnp.where(lc == ln, jnp.abs(pc - pn), fzero)
                b = jnp.minimum(
                    (v * jnp.float32(BINV)).astype(jnp.int32), NB - 1)
                m = (v > fzero) & mcol[dj]
                if di > PAD:
                    m = m & mrow[di]
                staged.append((v + v, b, m))
            for v2, b, m in staged:
                plsc.addupdate_scatter(cnt_v, [b], twos, mask=m)
                plsc.addupdate_scatter(sum_v, [b], v2, mask=m)

        vpad = jnp.where(lc == fzero, jnp.abs(pc), fzero)
        bpad = jnp.minimum(
            (vpad * jnp.float32(BINV)).astype(jnp.int32), NB - 1)
        cin = (KS - jnp.maximum(PAD - colv, 0)
               - jnp.maximum(colv - (W - 1 - PAD), 0))
        rin = (KS - jnp.maximum(PAD - g, 0)
               - jnp.maximum(g - (H - 1 - PAD), 0))
        nout = (KK - rin * cin).astype(jnp.float32)
        mpad = (vpad > fzero) & (nout > fzero)
        plsc.addupdate_scatter(cnt_v, [bpad], nout, mask=mpad)
        plsc.addupdate_scatter(sum_v, [bpad], nout * vpad, mask=mpad)

    plsc.parallel_loop(0, BROWS * CHUNKS, unroll=2)(it)

    pltpu.sync_copy(cnt_v, out_hbm.at[wid, 0])
    pltpu.sync_copy(sum_v, out_hbm.at[wid, 1])


_sc_hist = functools.partial(
    pl.kernel,
    mesh=plsc.VectorSubcoreMesh(core_axis_name="c", subcore_axis_name="s"),
    compiler_params=pltpu.CompilerParams(
        use_tc_tiling_on_sc=False, needs_layout_passes=False),
    out_type=jax.ShapeDtypeStruct((32, 2, NB), jnp.float32),
    scratch_types=[
        pltpu.VMEM((BROWS + 2 * PAD, PW), jnp.float32),
        pltpu.VMEM((BROWS + 2 * PAD, PW), jnp.float32),
        pltpu.VMEM((NB,), jnp.float32),
        pltpu.VMEM((NB,), jnp.float32),
    ],
)(_sc_hist_body)


@jax.jit
def kernel(pred, label):
    pp = jnp.pad(pred[:, 0], ((0, 0), (PAD, PAD), (PAD, PW - W - PAD)))
    lp = jnp.pad(label[:, 0], ((0, 0), (PAD, PAD), (PAD, PW - W - PAD)))
    hists = _sc_hist(pp, lp)
    hist = hists.reshape(2, 2, NBLK, 2, NB).sum(axis=2).reshape(NSAMP, 2, NB)
    cnt = hist[:, 0]
    vsum = hist[:, 1]
    kf = jnp.float32(TOP_NUM)
    c_incl = jnp.cumsum(cnt[:, ::-1], axis=1)[:, ::-1]
    s_incl = jnp.cumsum(vsum[:, ::-1], axis=1)[:, ::-1]
    c_excl = c_incl - cnt
    s_excl = s_incl - vsum
    bidx = jnp.arange(NB)[None, :]
    bstar = jnp.max(jnp.where(c_incl >= kf, bidx, 0), axis=1)
    pick = jax.vmap(lambda a, i: a[i])
    c_above = pick(c_excl, bstar)
    s_above = pick(s_excl, bstar)
    c_b = pick(cnt, bstar)
    s_b = pick(vsum, bstar)
    frac = jnp.where(c_b > 0, jnp.minimum((kf - c_above) / c_b, 1.0), 0.0)
    topk = s_above + frac * s_b
    return jnp.sum(topk) / jnp.float32(NSAMP * TOP_NUM)


# speedup vs baseline: 1.5452x; 1.5452x over previous
import functools

import jax
import jax.numpy as jnp
from jax import lax
from jax.experimental import pallas as pl
from jax.experimental.pallas import tpu as pltpu
from jax.experimental.pallas import tpu_sc as plsc

KS = 7
PAD = KS // 2
H = 384
W = 384
KK = KS * KS
TOP_NUM = (H * W * KK) // 10
NB = 2048
BINV = 32.0
NSAMP = 4
NBLK = 8
BROWS = H // NBLK
PH = H + 2 * PAD
PW = 392
CHUNKS = W // 16


def _sc_hist_body(pred_hbm, label_hbm, out_hbm, pv, lv, cnt_v, sum_v):
    cid = lax.axis_index("c")
    sid = lax.axis_index("s")
    wid = cid * 16 + sid
    samp = cid * 2 + sid // 8
    block = sid % 8
    row0 = BROWS * block

    pltpu.sync_copy(pred_hbm.at[samp, pl.ds(row0, BROWS + 2 * PAD)], pv)
    pltpu.sync_copy(label_hbm.at[samp, pl.ds(row0, BROWS + 2 * PAD)], lv)

    zeros16 = jnp.zeros((16,), jnp.float32)

    def zinit(i, carry):
        cnt_v[pl.ds(i * 16, 16)] = zeros16
        sum_v[pl.ds(i * 16, 16)] = zeros16
        return carry

    lax.fori_loop(0, NB // 16, zinit, 0)

    ones = jnp.ones((16,), jnp.float32)
    fzero = jnp.float32(0.0)

    offs = [(di, dj) for di in range(PAD + 1, KS) for dj in range(KS)]
    offs += [(PAD, dj) for dj in range(PAD + 1, KS)]
    GRP = 12
    twos = jnp.full((16,), 2.0, jnp.float32)
    lane = lax.iota(jnp.int32, 16)

    def it(t, carry):
        y = t // CHUNKS
        c0 = (t % CHUNKS) * 16
        g = BROWS * block + y
        colv = lane + c0
        pc = pv[y + PAD, pl.ds(c0 + PAD, 16)]
        lc = lv[y + PAD, pl.ds(c0 + PAD, 16)]

        mcol = {dj: ((colv + (dj - PAD)) >= 0) & ((colv + (dj - PAD)) < W)
                for dj in range(KS)}
        mrow = {di: (g + (di - PAD)) < H for di in range(PAD + 1, KS)}

        for g0 in range(0, len(offs), GRP):
            staged = []
            for di, dj in offs[g0:g0 + GRP]:
                pn = pv[y + di, pl.ds(c0 + dj, 16)]
                ln = lv[y + di, pl.ds(c0 + dj, 16)]
                v = jnp.where(lc == ln, jnp.abs(pc - pn), fzero)
                b = jnp.minimum(
                    (v * jnp.float32(BINV)).astype(jnp.int32), NB - 1)
                m = (v > fzero) & mcol[dj]
                if di > PAD:
                    m = m & mrow[di]
                staged.append((v + v, b, m))
            for v2, b, m in staged:
                plsc.addupdate_scatter(cnt_v, [b], twos, mask=m)
                plsc.addupdate_scatter(sum_v, [b], v2, mask=m)

        vpad = jnp.where(lc == fzero, jnp.abs(pc), fzero)
        bpad = jnp.minimum(
            (vpad * jnp.float32(BINV)).astype(jnp.int32), NB - 1)
        cin = (KS - jnp.maximum(PAD - colv, 0)
               - jnp.maximum(colv - (W - 1 - PAD), 0))
        rin = (KS - jnp.maximum(PAD - g, 0)
               - jnp.maximum(g - (H - 1 - PAD), 0))
        nout = (KK - rin * cin).astype(jnp.float32)
        mpad = (vpad > fzero) & (nout > fzero)
        plsc.addupdate_scatter(cnt_v, [bpad], nout, mask=mpad)
        plsc.addupdate_scatter(sum_v, [bpad], nout * vpad, mask=mpad)
        return carry

    lax.fori_loop(0, BROWS * CHUNKS, it, 0)

    pltpu.sync_copy(cnt_v, out_hbm.at[wid, 0])
    pltpu.sync_copy(sum_v, out_hbm.at[wid, 1])


_sc_hist = functools.partial(
    pl.kernel,
    mesh=plsc.VectorSubcoreMesh(core_axis_name="c", subcore_axis_name="s"),
    compiler_params=pltpu.CompilerParams(
        use_tc_tiling_on_sc=False, needs_layout_passes=False),
    out_type=jax.ShapeDtypeStruct((32, 2, NB), jnp.float32),
    scratch_types=[
        pltpu.VMEM((BROWS + 2 * PAD, PW), jnp.float32),
        pltpu.VMEM((BROWS + 2 * PAD, PW), jnp.float32),
        pltpu.VMEM((NB,), jnp.float32),
        pltpu.VMEM((NB,), jnp.float32),
    ],
)(_sc_hist_body)


@jax.jit
def kernel(pred, label):
    pp = jnp.pad(pred[:, 0], ((0, 0), (PAD, PAD), (PAD, PW - W - PAD)))
    lp = jnp.pad(label[:, 0], ((0, 0), (PAD, PAD), (PAD, PW - W - PAD)))
    hists = _sc_hist(pp, lp)
    hist = hists.reshape(2, 2, NBLK, 2, NB).sum(axis=2).reshape(NSAMP, 2, NB)
    cnt = hist[:, 0]
    vsum = hist[:, 1]
    kf = jnp.float32(TOP_NUM)
    c_incl = jnp.cumsum(cnt[:, ::-1], axis=1)[:, ::-1]
    s_incl = jnp.cumsum(vsum[:, ::-1], axis=1)[:, ::-1]
    c_excl = c_incl - cnt
    s_excl = s_incl - vsum
    bidx = jnp.arange(NB)[None, :]
    bstar = jnp.max(jnp.where(c_incl >= kf, bidx, 0), axis=1)
    pick = jax.vmap(lambda a, i: a[i])
    c_above = pick(c_excl, bstar)
    s_above = pick(s_excl, bstar)
    c_b = pick(cnt, bstar)
    s_b = pick(vsum, bstar)
    frac = jnp.where(c_b > 0, jnp.minimum((kf - c_above) / c_b, 1.0), 0.0)
    topk = s_above + frac * s_b
    return jnp.sum(topk) / jnp.float32(NSAMP * TOP_NUM)


# speedup vs baseline: 1.7032x; 1.1022x over previous
import functools

import jax
import jax.numpy as jnp
from jax import lax
from jax.experimental import pallas as pl
from jax.experimental.pallas import tpu as pltpu
from jax.experimental.pallas import tpu_sc as plsc

KS = 7
PAD = KS // 2
H = 384
W = 384
KK = KS * KS
TOP_NUM = (H * W * KK) // 10
NB = 2048
BINV = 32.0
NSAMP = 4
NBLK = 8
BROWS = H // NBLK
PH = H + 2 * PAD
PW = 392
CHUNKS = W // 16


def _sc_hist_body(pred_hbm, label_hbm, out_hbm, pv, lv, cnt_v, sum_v):
    cid = lax.axis_index("c")
    sid = lax.axis_index("s")
    wid = cid * 16 + sid
    samp = cid * 2 + sid // 8
    block = sid % 8
    row0 = BROWS * block

    pltpu.sync_copy(pred_hbm.at[samp, pl.ds(row0, BROWS + 2 * PAD)], pv)
    pltpu.sync_copy(label_hbm.at[samp, pl.ds(row0, BROWS + 2 * PAD)], lv)

    zeros16 = jnp.zeros((16,), jnp.float32)

    def zinit(i, carry):
        cnt_v[pl.ds(i * 16, 16)] = zeros16
        sum_v[pl.ds(i * 16, 16)] = zeros16
        return carry

    lax.fori_loop(0, NB // 16, zinit, 0)

    ones = jnp.ones((16,), jnp.float32)
    fzero = jnp.float32(0.0)

    offs = [(di, dj) for di in range(PAD + 1, KS) for dj in range(KS)]
    offs += [(PAD, dj) for dj in range(PAD + 1, KS)]
    GRP = 12
    twos = jnp.full((16,), 2.0, jnp.float32)
    lane = lax.iota(jnp.int32, 16)

    vmax = jnp.float32((NB - 1) / BINV)

    def chunk_it(ci, carry):
        c0 = ci * 16
        colv = lane + c0
        mcol = {dj: ((colv + (dj - PAD)) >= 0) & ((colv + (dj - PAD)) < W)
                for dj in range(KS)}
        cin = (KS - jnp.maximum(PAD - colv, 0)
               - jnp.maximum(colv - (W - 1 - PAD), 0))

        def row_it(y, carry2):
            g = BROWS * block + y
            pc = pv[y + PAD, pl.ds(c0 + PAD, 16)]
            lc = lv[y + PAD, pl.ds(c0 + PAD, 16)]
            mrow = {di: (g + (di - PAD)) < H for di in range(PAD + 1, KS)}

            for g0 in range(0, len(offs), GRP):
                staged = []
                for di, dj in offs[g0:g0 + GRP]:
                    pn = pv[y + di, pl.ds(c0 + dj, 16)]
                    ln = lv[y + di, pl.ds(c0 + dj, 16)]
                    v = jnp.where(lc == ln, jnp.abs(pc - pn), fzero)
                    b = (jnp.minimum(v, vmax)
                         * jnp.float32(BINV)).astype(jnp.int32)
                    m = (v > fzero) & mcol[dj]
                    if di > PAD:
                        m = m & mrow[di]
                    staged.append((v + v, b, m))
                for v2, b, m in staged:
                    plsc.addupdate_scatter(cnt_v, [b], twos, mask=m)
                    plsc.addupdate_scatter(sum_v, [b], v2, mask=m)

            vpad = jnp.where(lc == fzero, jnp.abs(pc), fzero)
            bpad = (jnp.minimum(vpad, vmax)
                    * jnp.float32(BINV)).astype(jnp.int32)
            rin = (KS - jnp.maximum(PAD - g, 0)
                   - jnp.maximum(g - (H - 1 - PAD), 0))
            nout = (KK - rin * cin).astype(jnp.float32)
            mpad = (vpad > fzero) & (nout > fzero)
            plsc.addupdate_scatter(cnt_v, [bpad], nout, mask=mpad)
            plsc.addupdate_scatter(sum_v, [bpad], nout * vpad, mask=mpad)
            return carry2

        lax.fori_loop(0, BROWS, row_it, 0)
        return carry

    lax.fori_loop(0, CHUNKS, chunk_it, 0)

    pltpu.sync_copy(cnt_v, out_hbm.at[wid, 0])
    pltpu.sync_copy(sum_v, out_hbm.at[wid, 1])


_sc_hist = functools.partial(
    pl.kernel,
    mesh=plsc.VectorSubcoreMesh(core_axis_name="c", subcore_axis_name="s"),
    compiler_params=pltpu.CompilerParams(
        use_tc_tiling_on_sc=False, needs_layout_passes=False),
    out_type=jax.ShapeDtypeStruct((32, 2, NB), jnp.float32),
    scratch_types=[
        pltpu.VMEM((BROWS + 2 * PAD, PW), jnp.float32),
        pltpu.VMEM((BROWS + 2 * PAD, PW), jnp.float32),
        pltpu.VMEM((NB,), jnp.float32),
        pltpu.VMEM((NB,), jnp.float32),
    ],
)(_sc_hist_body)


@jax.jit
def kernel(pred, label):
    pp = jnp.pad(pred[:, 0], ((0, 0), (PAD, PAD), (PAD, PW - W - PAD)))
    lp = jnp.pad(label[:, 0], ((0, 0), (PAD, PAD), (PAD, PW - W - PAD)))
    hists = _sc_hist(pp, lp)
    hist = hists.reshape(2, 2, NBLK, 2, NB).sum(axis=2).reshape(NSAMP, 2, NB)
    cnt = hist[:, 0]
    vsum = hist[:, 1]
    kf = jnp.float32(TOP_NUM)
    c_incl = jnp.cumsum(cnt[:, ::-1], axis=1)[:, ::-1]
    s_incl = jnp.cumsum(vsum[:, ::-1], axis=1)[:, ::-1]
    c_excl = c_incl - cnt
    s_excl = s_incl - vsum
    bidx = jnp.arange(NB)[None, :]
    bstar = jnp.max(jnp.where(c_incl >= kf, bidx, 0), axis=1)
    pick = jax.vmap(lambda a, i: a[i])
    c_above = pick(c_excl, bstar)
    s_above = pick(s_excl, bstar)
    c_b = pick(cnt, bstar)
    s_b = pick(vsum, bstar)
    frac = jnp.where(c_b > 0, jnp.minimum((kf - c_above) / c_b, 1.0), 0.0)
    topk = s_above + frac * s_b
    return jnp.sum(topk) / jnp.float32(NSAMP * TOP_NUM)
